# Initial kernel scaffold; baseline (speedup 1.0000x reference)
import jax
import jax.numpy as jnp
from jax.experimental import pallas as pl


def kernel(Q, K, V, W_sk, b_sk, W_sv, b_sv, W_sq, b_sq, W_pk, b_pk, W_pq, b_pq, W_qp, b_qp, W_kp, b_kp):
    raise NotImplementedError("write your pallas kernel here")


# speedup vs baseline: 17.4235x; 17.4235x over previous
import math

import jax
import jax.numpy as jnp
from jax.experimental import pallas as pl

_B, _H, _L, _DK = 4, 32, 4096, 64
_LK = 4096
_NC = 3
_LOG_L = 8
_LOG_LK = 8


def _main_kernel(q_ref, k_ref, v_ref, wsk_ref, bsk_ref, wsv_ref, bsv_ref,
                 wsq_ref, bsq_ref, wpk_ref, wpq_ref,
                 ctx_ref, zq_ref, zk_ref):
    b = pl.program_id(0)
    h = pl.program_id(1)

    @pl.when(jnp.logical_and(b == 0, h == 0))
    def _init():
        zq_ref[...] = jnp.zeros_like(zq_ref)
        zk_ref[...] = jnp.zeros_like(zk_ref)

    q = q_ref[0, 0]
    k = k_ref[0, 0]
    v = v_ref[0, 0]

    ks = jnp.dot(wsk_ref[...], k, preferred_element_type=jnp.float32) + bsk_ref[...]
    qs = jnp.dot(wsq_ref[...], q, preferred_element_type=jnp.float32) + bsq_ref[...]
    vs = jnp.dot(wsv_ref[...], v, preferred_element_type=jnp.float32) + bsv_ref[...]

    vm = jnp.mean(vs, axis=0, keepdims=True)
    ctx_ref[0, 0] = jnp.broadcast_to(vm, (_L, _DK))

    contrib_k = jnp.concatenate(
        [jnp.sum(ks * wpk_ref[c, 0], keepdims=True) for c in range(_NC)],
        axis=1)
    contrib_q = jnp.concatenate(
        [jnp.sum(qs * wpq_ref[c, 0], keepdims=True) for c in range(_NC)],
        axis=1)
    onehot = (jax.lax.broadcasted_iota(jnp.int32, (_B, 1), 0) == b
              ).astype(jnp.float32)
    zq_ref[...] += onehot * contrib_q
    zk_ref[...] += onehot * contrib_k


def _loss_kernel(zq_ref, zk_ref, bpq_ref, bpk_ref, wqp_ref, bqp_ref,
                 wkp_ref, bkp_ref, loss_ref):
    cqp = jnp.maximum(zq_ref[...] + bpq_ref[...], 0.0)
    ckp = jnp.maximum(zk_ref[...] + bpk_ref[...], 0.0)
    logit_q = jnp.dot(cqp, wqp_ref[...].T,
                      preferred_element_type=jnp.float32) + bqp_ref[...]
    logit_k = jnp.dot(ckp, wkp_ref[...].T,
                      preferred_element_type=jnp.float32) + bkp_ref[...]
    cluster_q = jax.nn.softmax(logit_q, axis=-1)
    cluster_k = jax.nn.softmax(logit_k, axis=-1)
    mu = jnp.mean(cluster_q, axis=0, keepdims=True)
    mk = jnp.mean(cluster_k, axis=0, keepdims=True)
    var = jnp.sum((cluster_k - mk) ** 2, axis=0, keepdims=True) / (_B - 1)
    sigma = jax.nn.softplus(jnp.sqrt(var))
    ll = (-0.5 * ((cluster_k - mu) / sigma) ** 2 - jnp.log(sigma)
          - 0.5 * math.log(2.0 * math.pi))
    lsm = jax.nn.log_softmax(cluster_q, axis=-1)
    ce_terms = jnp.sum(-cluster_q * lsm, axis=-1, keepdims=True)
    loss_ref[...] = (jnp.mean(ce_terms, axis=0, keepdims=True)
                     - jnp.mean(ll, keepdims=True))


def kernel(Q, K, V, W_sk, b_sk, W_sv, b_sv, W_sq, b_sq, W_pk, b_pk,
           W_pq, b_pq, W_qp, b_qp, W_kp, b_kp):
    qkv_spec = pl.BlockSpec((1, 1, _L, _DK), lambda b, h: (b, h, 0, 0))
    w_spec = pl.BlockSpec((_LOG_LK, _LK), lambda b, h: (0, 0))
    bias_spec = pl.BlockSpec((_LOG_LK, 1), lambda b, h: (0, 0))
    wp_spec = pl.BlockSpec((_NC, 1, _LOG_LK, _DK), lambda b, h: (0, h, 0, 0))
    acc_spec = pl.BlockSpec((_B, _NC), lambda b, h: (0, 0))

    ctx, zq, zk = pl.pallas_call(
        _main_kernel,
        grid=(_B, _H),
        in_specs=[qkv_spec, qkv_spec, qkv_spec,
                  w_spec, bias_spec, w_spec, bias_spec, w_spec, bias_spec,
                  wp_spec, wp_spec],
        out_specs=[qkv_spec, acc_spec, acc_spec],
        out_shape=[
            jax.ShapeDtypeStruct((_B, _H, _L, _DK), jnp.float32),
            jax.ShapeDtypeStruct((_B, _NC), jnp.float32),
            jax.ShapeDtypeStruct((_B, _NC), jnp.float32),
        ],
    )(Q, K, V,
      W_sk, b_sk.reshape(_LOG_LK, 1), W_sv, b_sv.reshape(_LOG_LK, 1),
      W_sq, b_sq.reshape(_LOG_L, 1),
      W_pk.reshape(_NC, _H, _LOG_LK, _DK),
      W_pq.reshape(_NC, _H, _LOG_L, _DK))

    loss = pl.pallas_call(
        _loss_kernel,
        out_shape=jax.ShapeDtypeStruct((1, 1), jnp.float32),
    )(zq, zk, b_pq.reshape(1, _NC), b_pk.reshape(1, _NC),
      W_qp, b_qp.reshape(1, _NC), W_kp, b_kp.reshape(1, _NC))

    return ctx, loss.reshape(())
